# speedup vs baseline: 2.3064x; 2.3064x over previous
import functools

import jax
import jax.numpy as jnp
from jax import lax
from jax.experimental import pallas as pl
from jax.experimental.pallas import tpu as pltpu
from jax.experimental.pallas import tpu_sc as plsc

_MAXD = 512
_D = 128
_N = 100000
_NC = 2
_NS = 16
_NW = _NC * _NS
_BPW = 3128
_NPAD = _NW * _BPW
_CH = 128
_NFULL = 23
_IDXBUF = 3136

_mesh = plsc.VectorSubcoreMesh(core_axis_name="c", subcore_axis_name="s")


@functools.partial(
    pl.kernel,
    out_type=jax.ShapeDtypeStruct((_N, _D), jnp.float32),
    mesh=_mesh,
    scratch_types=[
        pltpu.VMEM((_IDXBUF,), jnp.int32),
        pltpu.VMEM((_CH, _D), jnp.float32),
        pltpu.SemaphoreType.DMA,
    ],
)
def _degree_gather(deg_hbm, table_hbm, out_hbm, idx_v, rows_v, sem):
    wid = lax.axis_index("s") * _NC + lax.axis_index("c")
    base = wid * _BPW

    pltpu.sync_copy(deg_hbm.at[pl.ds(base, _BPW)], idx_v.at[pl.ds(0, _BPW)])

    def clamp_body(i, carry):
        off = jnp.minimum(i * 16, _BPW - 16)
        v = idx_v[pl.ds(off, 16)]
        idx_v[pl.ds(off, 16)] = jnp.minimum(jnp.maximum(v, 0), _MAXD - 1)
        return carry

    lax.fori_loop(0, _BPW // 16 + 1, clamp_body, 0)

    def chunk(off, nrows):
        pltpu.async_copy(
            table_hbm.at[idx_v.at[pl.ds(off, nrows)]],
            rows_v.at[pl.ds(0, nrows)],
            sem,
        ).wait()
        pltpu.sync_copy(
            rows_v.at[pl.ds(0, nrows)],
            out_hbm.at[pl.ds(base + off, nrows)],
        )

    def chunk_body(c, carry):
        chunk(c * _CH, _CH)
        return carry

    lax.fori_loop(0, _NFULL, chunk_body, 0)

    @pl.when(wid < _NW - 1)
    def _():
        chunk(_NFULL * _CH, _CH)
        chunk((_NFULL + 1) * _CH, _BPW - (_NFULL + 1) * _CH)

    @pl.when(wid == _NW - 1)
    def _():
        chunk(_NFULL * _CH, 88)


def kernel(degrees, degree_embedding):
    deg = degrees.astype(jnp.int32)
    deg_pad = jnp.pad(deg, (0, _NPAD - _N))
    return _degree_gather(deg_pad, degree_embedding)


# speedup vs baseline: 2.3872x; 1.0350x over previous
import functools

import jax
import jax.numpy as jnp
from jax import lax
from jax.experimental import pallas as pl
from jax.experimental.pallas import tpu as pltpu
from jax.experimental.pallas import tpu_sc as plsc

_MAXD = 512
_D = 128
_N = 100000
_NC = 2
_NS = 16
_NW = _NC * _NS
_BPW = 3128
_NPAD = _NW * _BPW
_CH = 128
_NBUF = 4
_NGRP = 6
_TAIL = _BPW - 24 * _CH
_LAST = 88
_IDXBUF = 3136

_mesh = plsc.VectorSubcoreMesh(core_axis_name="c", subcore_axis_name="s")


@functools.partial(
    pl.kernel,
    out_type=jax.ShapeDtypeStruct((_N, _D), jnp.float32),
    mesh=_mesh,
    scratch_types=[
        pltpu.VMEM((_IDXBUF,), jnp.int32),
        pltpu.VMEM((_NBUF, _CH, _D), jnp.float32),
        [pltpu.SemaphoreType.DMA] * _NBUF,
        [pltpu.SemaphoreType.DMA] * _NBUF,
    ],
)
def _degree_gather(deg_hbm, table_hbm, out_hbm, idx_v, rows_v, gsem, wsem):
    wid = lax.axis_index("s") * _NC + lax.axis_index("c")
    base = wid * _BPW
    last = wid == _NW - 1

    pltpu.sync_copy(deg_hbm.at[pl.ds(base, _BPW)], idx_v.at[pl.ds(0, _BPW)])

    def clamp_body(i, carry):
        off = jnp.minimum(i * 16, _BPW - 16)
        v = idx_v[pl.ds(off, 16)]
        idx_v[pl.ds(off, 16)] = jnp.minimum(jnp.maximum(v, 0), _MAXD - 1)
        return carry

    lax.fori_loop(0, _BPW // 16 + 1, clamp_body, 0)

    def gather(off, nrows, j):
        return pltpu.make_async_copy(
            table_hbm.at[idx_v.at[pl.ds(off, nrows)]],
            rows_v.at[j].at[pl.ds(0, nrows)],
            gsem[j],
        )

    def write(off, nrows, j):
        return pltpu.make_async_copy(
            rows_v.at[j].at[pl.ds(0, nrows)],
            out_hbm.at[pl.ds(base + off, nrows)],
            wsem[j],
        )

    def write_desc(nrows, j):
        return pltpu.make_async_copy(
            rows_v.at[j].at[pl.ds(0, nrows)],
            out_hbm.at[pl.ds(base, nrows)],
            wsem[j],
        )

    def group(g, carry):
        for j in range(_NBUF):
            @pl.when(g > 0)
            def _():
                write_desc(_CH, j).wait()

            gather((g * _NBUF + j) * _CH, _CH, j).start()
        for j in range(_NBUF):
            gather(0, _CH, j).wait()
            c = g * _NBUF + j
            if j == _NBUF - 1:
                full = jnp.logical_or(c < _NGRP * _NBUF - 1, ~last)

                @pl.when(full)
                def _():
                    write(c * _CH, _CH, j).start()

                @pl.when(~full)
                def _():
                    write(c * _CH, _LAST, j).start()
            else:
                write(c * _CH, _CH, j).start()
        return carry

    lax.fori_loop(0, _NGRP, group, 0)

    for j in range(_NBUF - 1):
        write_desc(_CH, j).wait()

    @pl.when(~last)
    def _():
        write_desc(_CH, _NBUF - 1).wait()
        tail = gather(24 * _CH, _TAIL, 0)
        tail.start()
        tail.wait()
        pltpu.sync_copy(
            rows_v.at[0].at[pl.ds(0, _TAIL)],
            out_hbm.at[pl.ds(base + 24 * _CH, _TAIL)],
        )

    @pl.when(last)
    def _():
        write_desc(_LAST, _NBUF - 1).wait()


def kernel(degrees, degree_embedding):
    deg = degrees.astype(jnp.int32)
    deg_pad = jnp.pad(deg, (0, _NPAD - _N))
    return _degree_gather(deg_pad, degree_embedding)


# speedup vs baseline: 5.6457x; 2.3650x over previous
import functools

import jax
import jax.numpy as jnp
from jax import lax
from jax.experimental import pallas as pl
from jax.experimental.pallas import tpu as pltpu
from jax.experimental.pallas import tpu_sc as plsc

_MAXD = 512
_D = 128
_N = 100000
_NC = 2
_NS = 16
_NW = _NC * _NS
_BPW = 3128
_NPAD = _NW * _BPW
_CH = 128
_NBUF = 4
_NGRP = 6
_TAIL = _BPW - 24 * _CH
_LAST = 88
_IDXBUF = 3136

_mesh = plsc.VectorSubcoreMesh(core_axis_name="c", subcore_axis_name="s")


@functools.partial(
    pl.kernel,
    out_type=jax.ShapeDtypeStruct((_N, _D), jnp.float32),
    mesh=_mesh,
    scratch_types=[
        pltpu.VMEM((_IDXBUF,), jnp.int32),
        pltpu.VMEM((_NBUF, _CH, _D), jnp.float32),
        pltpu.VMEM_SHARED((_MAXD, _D), jnp.float32),
        [pltpu.SemaphoreType.DMA] * _NBUF,
        [pltpu.SemaphoreType.DMA] * _NBUF,
        pltpu.SemaphoreType.DMA,
    ],
)
def _degree_gather(deg_hbm, table_hbm, out_hbm, idx_v, rows_v, table_sp,
                   gsem, wsem, tsem):
    sid = lax.axis_index("s")
    wid = sid * _NC + lax.axis_index("c")
    base = wid * _BPW
    last = wid == _NW - 1

    @pl.when(sid == 0)
    def _():
        pltpu.make_async_copy(table_hbm, table_sp, tsem).start()

    pltpu.sync_copy(deg_hbm.at[pl.ds(base, _BPW)], idx_v.at[pl.ds(0, _BPW)])

    def clamp_body(i, carry):
        off = jnp.minimum(i * 16, _BPW - 16)
        v = idx_v[pl.ds(off, 16)]
        idx_v[pl.ds(off, 16)] = jnp.minimum(jnp.maximum(v, 0), _MAXD - 1)
        return carry

    lax.fori_loop(0, _BPW // 16 + 1, clamp_body, 0)

    @pl.when(sid == 0)
    def _():
        pltpu.make_async_copy(table_hbm, table_sp, tsem).wait()

    plsc.subcore_barrier()

    def gather(off, nrows, j):
        return pltpu.make_async_copy(
            table_sp.at[idx_v.at[pl.ds(off, nrows)]],
            rows_v.at[j].at[pl.ds(0, nrows)],
            gsem[j],
        )

    def write(off, nrows, j):
        return pltpu.make_async_copy(
            rows_v.at[j].at[pl.ds(0, nrows)],
            out_hbm.at[pl.ds(base + off, nrows)],
            wsem[j],
        )

    def write_desc(nrows, j):
        return pltpu.make_async_copy(
            rows_v.at[j].at[pl.ds(0, nrows)],
            out_hbm.at[pl.ds(base, nrows)],
            wsem[j],
        )

    def group(g, carry):
        for j in range(_NBUF):
            @pl.when(g > 0)
            def _():
                write_desc(_CH, j).wait()

            gather((g * _NBUF + j) * _CH, _CH, j).start()
        for j in range(_NBUF):
            gather(0, _CH, j).wait()
            c = g * _NBUF + j
            if j == _NBUF - 1:
                full = jnp.logical_or(c < _NGRP * _NBUF - 1, ~last)

                @pl.when(full)
                def _():
                    write(c * _CH, _CH, j).start()

                @pl.when(~full)
                def _():
                    write(c * _CH, _LAST, j).start()
            else:
                write(c * _CH, _CH, j).start()
        return carry

    lax.fori_loop(0, _NGRP, group, 0)

    for j in range(_NBUF - 1):
        write_desc(_CH, j).wait()

    @pl.when(~last)
    def _():
        write_desc(_CH, _NBUF - 1).wait()
        tail = gather(24 * _CH, _TAIL, 0)
        tail.start()
        tail.wait()
        pltpu.sync_copy(
            rows_v.at[0].at[pl.ds(0, _TAIL)],
            out_hbm.at[pl.ds(base + 24 * _CH, _TAIL)],
        )

    @pl.when(last)
    def _():
        write_desc(_LAST, _NBUF - 1).wait()


def kernel(degrees, degree_embedding):
    deg = degrees.astype(jnp.int32)
    deg_pad = jnp.pad(deg, (0, _NPAD - _N))
    return _degree_gather(deg_pad, degree_embedding)


# speedup vs baseline: 5.7252x; 1.0141x over previous
import functools

import jax
import jax.numpy as jnp
from jax import lax
from jax.experimental import pallas as pl
from jax.experimental.pallas import tpu as pltpu
from jax.experimental.pallas import tpu_sc as plsc

_MAXD = 512
_D = 128
_N = 100000
_NC = 2
_NS = 16
_NW = _NC * _NS
_BPW = 3128
_NPAD = _NW * _BPW
_CH = 128
_NBUF = 4
_NGRP = 6
_TAIL = _BPW - 24 * _CH
_LAST = 88
_IDXBUF = 3136

_mesh = plsc.VectorSubcoreMesh(core_axis_name="c", subcore_axis_name="s")


@functools.partial(
    pl.kernel,
    out_type=jax.ShapeDtypeStruct((_N, _D), jnp.float32),
    mesh=_mesh,
    scratch_types=[
        pltpu.VMEM((_IDXBUF,), jnp.int32),
        pltpu.VMEM((_NBUF, _CH, _D), jnp.float32),
        pltpu.VMEM_SHARED((_MAXD, _D), jnp.float32),
        [pltpu.SemaphoreType.DMA] * _NBUF,
        [pltpu.SemaphoreType.DMA] * _NBUF,
        pltpu.SemaphoreType.DMA,
    ],
)
def _degree_gather(deg_hbm, table_hbm, out_hbm, idx_v, rows_v, table_sp,
                   gsem, wsem, tsem):
    sid = lax.axis_index("s")
    wid = sid * _NC + lax.axis_index("c")
    base = wid * _BPW
    last = wid == _NW - 1

    @pl.when(sid == 0)
    def _():
        pltpu.make_async_copy(table_hbm, table_sp, tsem).start()

    @pl.when(~last)
    def _():
        pltpu.sync_copy(deg_hbm.at[pl.ds(base, _BPW)], idx_v.at[pl.ds(0, _BPW)])

    @pl.when(last)
    def _():
        pltpu.sync_copy(
            deg_hbm.at[pl.ds(base, _N - (_NW - 1) * _BPW)],
            idx_v.at[pl.ds(0, _N - (_NW - 1) * _BPW)],
        )

    def clamp_chunk(off):
        for k in range(_CH // 16):
            v = idx_v[pl.ds(off + k * 16, 16)]
            idx_v[pl.ds(off + k * 16, 16)] = (
                jnp.minimum(jnp.maximum(v, 0), _MAXD - 1))

    @pl.when(sid == 0)
    def _():
        pltpu.make_async_copy(table_hbm, table_sp, tsem).wait()

    plsc.subcore_barrier()

    def gather(off, nrows, j):
        return pltpu.make_async_copy(
            table_sp.at[idx_v.at[pl.ds(off, nrows)]],
            rows_v.at[j].at[pl.ds(0, nrows)],
            gsem[j],
        )

    def write(off, nrows, j):
        return pltpu.make_async_copy(
            rows_v.at[j].at[pl.ds(0, nrows)],
            out_hbm.at[pl.ds(base + off, nrows)],
            wsem[j],
        )

    def write_desc(nrows, j):
        return pltpu.make_async_copy(
            rows_v.at[j].at[pl.ds(0, nrows)],
            out_hbm.at[pl.ds(base, nrows)],
            wsem[j],
        )

    def group(g, carry):
        for j in range(_NBUF):
            clamp_chunk((g * _NBUF + j) * _CH)

            @pl.when(g > 0)
            def _():
                write_desc(_CH, j).wait()

            gather((g * _NBUF + j) * _CH, _CH, j).start()
        for j in range(_NBUF):
            gather(0, _CH, j).wait()
            c = g * _NBUF + j
            if j == _NBUF - 1:
                full = jnp.logical_or(c < _NGRP * _NBUF - 1, ~last)

                @pl.when(full)
                def _():
                    write(c * _CH, _CH, j).start()

                @pl.when(~full)
                def _():
                    write(c * _CH, _LAST, j).start()
            else:
                write(c * _CH, _CH, j).start()
        return carry

    lax.fori_loop(0, _NGRP, group, 0)

    for j in range(_NBUF - 1):
        write_desc(_CH, j).wait()

    @pl.when(~last)
    def _():
        write_desc(_CH, _NBUF - 1).wait()
        for k in range(4):
            off = 24 * _CH + k * 16
            v = idx_v[pl.ds(off, 16)]
            idx_v[pl.ds(off, 16)] = jnp.minimum(jnp.maximum(v, 0), _MAXD - 1)
        tail = gather(24 * _CH, _TAIL, 0)
        tail.start()
        tail.wait()
        pltpu.sync_copy(
            rows_v.at[0].at[pl.ds(0, _TAIL)],
            out_hbm.at[pl.ds(base + 24 * _CH, _TAIL)],
        )

    @pl.when(last)
    def _():
        write_desc(_LAST, _NBUF - 1).wait()


def kernel(degrees, degree_embedding):
    return _degree_gather(degrees.astype(jnp.int32), degree_embedding)


# speedup vs baseline: 5.7584x; 1.0058x over previous
import functools

import jax
import jax.numpy as jnp
from jax import lax
from jax.experimental import pallas as pl
from jax.experimental.pallas import tpu as pltpu
from jax.experimental.pallas import tpu_sc as plsc

_MAXD = 512
_D = 128
_N = 100000
_NC = 2
_NS = 16
_NW = _NC * _NS
_BPW = 3128
_CH = 128
_NBUF = 6
_NGRP = 4
_NPAIR = _NBUF // 2
_TAIL = _BPW - 24 * _CH
_LASTN = _N - (_NW - 1) * _BPW
_LAST = _LASTN - 23 * _CH
_IDXBUF = 3136

_mesh = plsc.VectorSubcoreMesh(core_axis_name="c", subcore_axis_name="s")


@functools.partial(
    pl.kernel,
    out_type=jax.ShapeDtypeStruct((_N, _D), jnp.float32),
    mesh=_mesh,
    scratch_types=[
        pltpu.VMEM((_IDXBUF,), jnp.int32),
        pltpu.VMEM((_NBUF * _CH, _D), jnp.float32),
        pltpu.VMEM_SHARED((_MAXD, _D), jnp.float32),
        [pltpu.SemaphoreType.DMA] * _NBUF,
        [pltpu.SemaphoreType.DMA] * _NPAIR,
        pltpu.SemaphoreType.DMA,
    ],
)
def _degree_gather(deg_hbm, table_hbm, out_hbm, idx_v, rows_v, table_sp,
                   gsem, wsem, tsem):
    sid = lax.axis_index("s")
    wid = sid * _NC + lax.axis_index("c")
    base = wid * _BPW
    last = wid == _NW - 1

    @pl.when(sid == 0)
    def _():
        pltpu.make_async_copy(table_hbm, table_sp, tsem).start()

    @pl.when(~last)
    def _():
        pltpu.sync_copy(deg_hbm.at[pl.ds(base, _BPW)], idx_v.at[pl.ds(0, _BPW)])

    @pl.when(last)
    def _():
        pltpu.sync_copy(
            deg_hbm.at[pl.ds(base, _LASTN)],
            idx_v.at[pl.ds(0, _LASTN)],
        )

    def clamp_chunk(off):
        for k in range(_CH // 16):
            v = idx_v[pl.ds(off + k * 16, 16)]
            idx_v[pl.ds(off + k * 16, 16)] = (
                jnp.minimum(jnp.maximum(v, 0), _MAXD - 1))

    @pl.when(sid == 0)
    def _():
        pltpu.make_async_copy(table_hbm, table_sp, tsem).wait()

    plsc.subcore_barrier()

    def gather(off, nrows, j):
        return pltpu.make_async_copy(
            table_sp.at[idx_v.at[pl.ds(off, nrows)]],
            rows_v.at[pl.ds(j * _CH, nrows)],
            gsem[j],
        )

    def write(off, nrows, p):
        return pltpu.make_async_copy(
            rows_v.at[pl.ds(2 * p * _CH, nrows)],
            out_hbm.at[pl.ds(base + off, nrows)],
            wsem[p],
        )

    def write_desc(nrows, p):
        return pltpu.make_async_copy(
            rows_v.at[pl.ds(2 * p * _CH, nrows)],
            out_hbm.at[pl.ds(base, nrows)],
            wsem[p],
        )

    def group(g, carry):
        for j in range(_NBUF):
            clamp_chunk((g * _NBUF + j) * _CH)
            if j % 2 == 0:
                @pl.when(g > 0)
                def _():
                    write_desc(2 * _CH, j // 2).wait()

            gather((g * _NBUF + j) * _CH, _CH, j).start()
        for p in range(_NPAIR):
            gather(0, _CH, 2 * p).wait()
            gather(0, _CH, 2 * p + 1).wait()
            c = g * _NBUF + 2 * p
            if p == _NPAIR - 1:
                full = jnp.logical_or(c + 1 < _NGRP * _NBUF - 1, ~last)

                @pl.when(full)
                def _():
                    write(c * _CH, 2 * _CH, p).start()

                @pl.when(~full)
                def _():
                    write(c * _CH, _CH + _LAST, p).start()
            else:
                write(c * _CH, 2 * _CH, p).start()
        return carry

    lax.fori_loop(0, _NGRP, group, 0)

    for p in range(_NPAIR - 1):
        write_desc(2 * _CH, p).wait()

    @pl.when(~last)
    def _():
        write_desc(2 * _CH, _NPAIR - 1).wait()
        for k in range(_TAIL // 16 + 1):
            off = 24 * _CH + k * 16
            v = idx_v[pl.ds(off, 16)]
            idx_v[pl.ds(off, 16)] = jnp.minimum(jnp.maximum(v, 0), _MAXD - 1)
        tail = gather(24 * _CH, _TAIL, 0)
        tail.start()
        tail.wait()
        pltpu.sync_copy(
            rows_v.at[pl.ds(0, _TAIL)],
            out_hbm.at[pl.ds(base + 24 * _CH, _TAIL)],
        )

    @pl.when(last)
    def _():
        write_desc(_CH + _LAST, _NPAIR - 1).wait()


def kernel(degrees, degree_embedding):
    return _degree_gather(degrees.astype(jnp.int32), degree_embedding)


# speedup vs baseline: 5.7734x; 1.0026x over previous
import functools

import jax
import jax.numpy as jnp
from jax import lax
from jax.experimental import pallas as pl
from jax.experimental.pallas import tpu as pltpu
from jax.experimental.pallas import tpu_sc as plsc

_MAXD = 512
_D = 128
_N = 100000
_NC = 2
_NS = 16
_NW = _NC * _NS
_BPW = 3128
_CH = 128
_NBUF = 6
_NGRP = 4
_NPAIR = _NBUF // 2
_TAIL = _BPW - 24 * _CH
_LASTN = _N - (_NW - 1) * _BPW
_LAST = _LASTN - 23 * _CH
_IDXBUF = 3136

_mesh = plsc.VectorSubcoreMesh(core_axis_name="c", subcore_axis_name="s")


@functools.partial(
    pl.kernel,
    out_type=jax.ShapeDtypeStruct((_N, _D), jnp.float32),
    mesh=_mesh,
    scratch_types=[
        pltpu.VMEM((_IDXBUF,), jnp.int32),
        pltpu.VMEM((_NBUF * _CH + _TAIL, _D), jnp.float32),
        pltpu.VMEM_SHARED((_MAXD, _D), jnp.float32),
        [pltpu.SemaphoreType.DMA] * _NBUF,
        [pltpu.SemaphoreType.DMA] * _NPAIR,
        pltpu.SemaphoreType.DMA,
    ],
)
def _degree_gather(deg_hbm, table_hbm, out_hbm, idx_v, rows_v, table_sp,
                   gsem, wsem, tsem):
    sid = lax.axis_index("s")
    wid = sid * _NC + lax.axis_index("c")
    base = wid * _BPW
    last = wid == _NW - 1

    @pl.when(sid == 0)
    def _():
        pltpu.make_async_copy(table_hbm, table_sp, tsem).start()

    @pl.when(~last)
    def _():
        pltpu.sync_copy(deg_hbm.at[pl.ds(base, _BPW)], idx_v.at[pl.ds(0, _BPW)])

    @pl.when(last)
    def _():
        pltpu.sync_copy(
            deg_hbm.at[pl.ds(base, _LASTN)],
            idx_v.at[pl.ds(0, _LASTN)],
        )

    def clamp_chunk(off):
        for k in range(_CH // 16):
            v = idx_v[pl.ds(off + k * 16, 16)]
            idx_v[pl.ds(off + k * 16, 16)] = (
                jnp.minimum(jnp.maximum(v, 0), _MAXD - 1))

    @pl.when(sid == 0)
    def _():
        pltpu.make_async_copy(table_hbm, table_sp, tsem).wait()

    plsc.subcore_barrier()

    def gather(off, nrows, j):
        return pltpu.make_async_copy(
            table_sp.at[idx_v.at[pl.ds(off, nrows)]],
            rows_v.at[pl.ds(j * _CH, nrows)],
            gsem[j],
        )

    def write(off, nrows, p):
        return pltpu.make_async_copy(
            rows_v.at[pl.ds(2 * p * _CH, nrows)],
            out_hbm.at[pl.ds(base + off, nrows)],
            wsem[p],
        )

    def write_desc(nrows, p):
        return pltpu.make_async_copy(
            rows_v.at[pl.ds(2 * p * _CH, nrows)],
            out_hbm.at[pl.ds(base, nrows)],
            wsem[p],
        )

    def tail_gather():
        return pltpu.make_async_copy(
            table_sp.at[idx_v.at[pl.ds(24 * _CH, _TAIL)]],
            rows_v.at[pl.ds(_NBUF * _CH, _TAIL)],
            tsem,
        )

    def group(g, carry):
        final = g == _NGRP - 1
        for j in range(_NBUF):
            clamp_chunk((g * _NBUF + j) * _CH)
            if j % 2 == 0:
                @pl.when(g > 0)
                def _():
                    write_desc(2 * _CH, j // 2).wait()

            gather((g * _NBUF + j) * _CH, _CH, j).start()

        @pl.when(jnp.logical_and(final, ~last))
        def _():
            for k in range(_TAIL // 16 + 1):
                off = 24 * _CH + k * 16
                v = idx_v[pl.ds(off, 16)]
                idx_v[pl.ds(off, 16)] = (
                    jnp.minimum(jnp.maximum(v, 0), _MAXD - 1))
            tail_gather().start()

        for p in range(_NPAIR):
            gather(0, _CH, 2 * p).wait()
            gather(0, _CH, 2 * p + 1).wait()
            c = g * _NBUF + 2 * p
            if p == _NPAIR - 1:
                @pl.when(~final)
                def _():
                    write(c * _CH, 2 * _CH, p).start()

                @pl.when(jnp.logical_and(final, ~last))
                def _():
                    tail_gather().wait()
                    write(c * _CH, 2 * _CH + _TAIL, p).start()

                @pl.when(jnp.logical_and(final, last))
                def _():
                    write(c * _CH, _CH + _LAST, p).start()
            else:
                write(c * _CH, 2 * _CH, p).start()
        return carry

    lax.fori_loop(0, _NGRP, group, 0)

    for p in range(_NPAIR - 1):
        write_desc(2 * _CH, p).wait()

    @pl.when(~last)
    def _():
        write_desc(2 * _CH + _TAIL, _NPAIR - 1).wait()

    @pl.when(last)
    def _():
        write_desc(_CH + _LAST, _NPAIR - 1).wait()


def kernel(degrees, degree_embedding):
    return _degree_gather(degrees.astype(jnp.int32), degree_embedding)
